# baseline (speedup 1.0000x reference)
import functools
import math

import jax
import jax.numpy as jnp
from jax import lax
from jax.experimental import pallas as pl
from jax.experimental.pallas import tpu as pltpu
from jax.experimental.pallas import tpu_sc as plsc

_B, _T, _D = 32, 16, 4096
_N = 5
_BW = 2.0
_L = 16
_WIN = 64
_PAD = 128
_ACC = _D + 2 * _PAD
_NEG_INF = float("-inf")
_BIG_I32 = 0x7FFFFFFF
_U = 16
_CAND = _D + _L


def cand_buf_slice(cref, cnt):
    return cref.at[pl.ds(cnt, _L)]


def _sc_kernel(x_hbm, out_hbm, row_v, acc_v, cand_a, cand_b, sem):
    c = lax.axis_index("c")
    s = lax.axis_index("s")
    w = s * 2 + c
    t = w // 2
    b_base = (w % 2) * 16

    iota = lax.iota(jnp.int32, _L)
    zeros = jnp.zeros((_L,), jnp.float32)

    def zero_body(i, _):
        acc_v[pl.ds(i * _L, _L)] = zeros
        return 0
    lax.fori_loop(0, _ACC // _L, zero_body, 0)

    coef = jnp.float32(1.0 / (_BW * math.sqrt(2.0 * math.pi)))
    table = []
    for jv in range(_WIN // _L):
        o = (iota + (jv * _L - 31)).astype(jnp.float32)
        z = o * jnp.float32(1.0 / _BW)
        table.append(coef * jnp.exp(jnp.float32(-0.5) * z * z))

    def fire_pair(jj):
        for r in range(2):
            b = b_base + jj * 2 + r
            off = ((jj % 2) * 2 + r) * _D
            pltpu.make_async_copy(
                x_hbm.at[b, t], row_v.at[pl.ds(off, _D)], sem).start()

    def drain_pair(jj):
        for r in range(2):
            b = b_base + jj * 2 + r
            off = ((jj % 2) * 2 + r) * _D
            pltpu.make_async_copy(
                x_hbm.at[b, t], row_v.at[pl.ds(off, _D)], sem).wait()

    fire_pair(0)

    minf_v = jnp.full((_L,), _NEG_INF, jnp.float32)
    big_v = jnp.full((_L,), _BIG_I32, jnp.int32)
    iota_u = [iota + u * _L for u in range(_U)]

    def fifth_lane_max(m):
        for _ in range(_N - 1):
            gm = jnp.max(m)
            m = jnp.where(m == gm, minf_v, m)
        return jnp.max(m)

    def select_row(cref, kcnt, o):
        ms = [minf_v] * _N
        is_ = [big_v] * _N
        nst = (kcnt + (_L - 1)) // _L

        def body(q, carry):
            ms = list(carry[:_N])
            is_ = list(carry[_N:])
            pos = q * _L + iota
            valid = pos < kcnt
            idx_raw = cref[pl.ds(q * _L, _L)]
            idx_safe = jnp.where(valid, idx_raw, 0)
            x = plsc.load_gather(row_v, [o + idx_safe])
            x = jnp.where(valid, x, minf_v)
            xi = jnp.where(valid, idx_raw, big_v)
            for k in range(_N):
                bsel = x > ms[k]
                nm = jnp.where(bsel, x, ms[k])
                ni = jnp.where(bsel, xi, is_[k])
                x = jnp.where(bsel, ms[k], x)
                xi = jnp.where(bsel, is_[k], xi)
                ms[k] = nm
                is_[k] = ni
            return tuple(ms) + tuple(is_)

        return lax.fori_loop(0, nst, body, tuple(ms) + tuple(is_))

    def extract_accumulate(state):
        ms = list(state[:_N])
        is_ = list(state[_N:])
        for _k in range(_N):
            gmax = jnp.max(ms[0])
            pm = ms[0] == gmax
            ibest = jnp.min(jnp.where(pm, is_[0], _BIG_I32))
            base = ibest + (_PAD - 31)
            for jv in range(_WIN // _L):
                idxv = base + (jv * _L) + iota
                plsc.addupdate_scatter(acc_v, [idxv], table[jv])
            pop = is_[0] == ibest
            for k in range(_N - 1):
                ms[k] = jnp.where(pop, ms[k + 1], ms[k])
                is_[k] = jnp.where(pop, is_[k + 1], is_[k])
            ms[_N - 1] = jnp.where(pop, jnp.full((_L,), _NEG_INF, jnp.float32),
                                   ms[_N - 1])
            is_[_N - 1] = jnp.where(pop, jnp.full((_L,), _BIG_I32, jnp.int32),
                                    is_[_N - 1])

    def pair_body(jj, _):
        drain_pair(jj)

        @pl.when(jj < 7)
        def _prefetch():
            fire_pair(jj + 1)

        slot_off = (jj % 2) * (2 * _D)
        oa = slot_off
        ob = slot_off + _D

        def tree_max(xs):
            vals = list(xs)
            while len(vals) > 1:
                vals = [jnp.maximum(vals[2 * i], vals[2 * i + 1])
                        for i in range(len(vals) // 2)]
            return vals[0]

        def p1(st, carry):
            ma, mb, cka, ckb = carry
            base = st * (_U * _L)
            ta = tree_max([row_v[pl.ds(oa + base + u * _L, _L)]
                           for u in range(_U)])
            tb = tree_max([row_v[pl.ds(ob + base + u * _L, _L)]
                           for u in range(_U)])
            ma = jnp.maximum(ma, ta)
            mb = jnp.maximum(mb, tb)
            lane = iota == st
            cka = jnp.where(lane, jnp.max(ta), cka)
            ckb = jnp.where(lane, jnp.max(tb), ckb)
            return ma, mb, cka, ckb

        ma, mb, cka, ckb = lax.fori_loop(
            0, _D // (_U * _L), p1, (minf_v, minf_v, minf_v, minf_v))
        ska, _sva = plsc.sort_key_val(ma, ma)
        skb, _svb = plsc.sort_key_val(mb, mb)
        t0a = jnp.squeeze(lax.slice(ska, (11,), (12,)))
        t0b = jnp.squeeze(lax.slice(skb, (11,), (12,)))
        hva = jnp.where(cka >= t0a, 1, 0)
        hvb = jnp.where(ckb >= t0b, 1, 0)

        def work(base, o, t0, cref, cnt):
            xs = [row_v[pl.ds(o + base + u * _L, _L)] for u in range(_U)]
            msks = [x >= t0 for x in xs]
            for u in range(_U):
                xi = iota_u[u] + base
                plsc.store_compressed(
                    cand_buf_slice(cref, cnt), xi, mask=msks[u])
                cnt = cnt + jnp.sum(msks[u], dtype=jnp.int32)
            return cnt

        ka = jnp.int32(0)
        kb = jnp.int32(0)
        for st in range(_D // (_U * _L)):
            base = st * (_U * _L)
            sel_a = jnp.squeeze(lax.slice(hva, (st,), (st + 1,))) != 0
            sel_b = jnp.squeeze(lax.slice(hvb, (st,), (st + 1,))) != 0
            ka = lax.cond(sel_a,
                          functools.partial(work, base, oa, t0a, cand_a),
                          lambda v: v, ka)
            kb = lax.cond(sel_b,
                          functools.partial(work, base, ob, t0b, cand_b),
                          lambda v: v, kb)

        sa = select_row(cand_a, ka, oa)
        sb = select_row(cand_b, kb, ob)
        extract_accumulate(sa)
        extract_accumulate(sb)
        return 0

    lax.fori_loop(0, 8, pair_body, 0)

    for cc in range(_D // 128):
        pltpu.make_async_copy(
            acc_v.at[pl.ds(_PAD + cc * 128, 128)],
            out_hbm.at[w, pl.ds(cc * 128, 128)], sem).start()
    for cc in range(_D // 128):
        pltpu.make_async_copy(
            acc_v.at[pl.ds(_PAD + cc * 128, 128)],
            out_hbm.at[w, pl.ds(cc * 128, 128)], sem).wait()


@jax.jit
def kernel(inputs):
    assert inputs.shape == (_B, _T, _D) and inputs.dtype == jnp.float32
    mesh = plsc.VectorSubcoreMesh(core_axis_name="c", subcore_axis_name="s")
    run = pl.kernel(
        _sc_kernel,
        out_type=jax.ShapeDtypeStruct((_B, _D), jnp.float32),
        mesh=mesh,
        scratch_types=[
            pltpu.VMEM((4 * _D,), jnp.float32),
            pltpu.VMEM((_ACC,), jnp.float32),
            pltpu.VMEM((_CAND,), jnp.int32),
            pltpu.VMEM((_CAND,), jnp.int32),
            pltpu.SemaphoreType.DMA,
        ],
        compiler_params=pltpu.CompilerParams(
            needs_layout_passes=False, use_tc_tiling_on_sc=True),
    )
    return run(inputs)


# speedup vs baseline: 1.3483x; 1.3483x over previous
import functools
import math

import jax
import jax.numpy as jnp
from jax import lax
from jax.experimental import pallas as pl
from jax.experimental.pallas import tpu as pltpu
from jax.experimental.pallas import tpu_sc as plsc

_B, _T, _D = 32, 16, 4096
_N = 5
_BW = 2.0
_L = 16
_WIN = 64
_PAD = 128
_ACC = _D + 2 * _PAD
_NEG_INF = float("-inf")
_BIG_I32 = 0x7FFFFFFF
_U = 16
_CAND = _D + _L


def cand_buf_slice(cref, cnt):
    return cref.at[pl.ds(cnt, _L)]


def _sc_kernel(x_hbm, out_hbm, row_v, acc_v, cand_a, cand_b, sem):
    c = lax.axis_index("c")
    s = lax.axis_index("s")
    w = s * 2 + c
    t = w // 2
    b_base = (w % 2) * 16

    iota = lax.iota(jnp.int32, _L)
    zeros = jnp.zeros((_L,), jnp.float32)

    def zero_body(i, _):
        acc_v[pl.ds(i * _L, _L)] = zeros
        return 0
    lax.fori_loop(0, _ACC // _L, zero_body, 0)

    coef = jnp.float32(1.0 / (_BW * math.sqrt(2.0 * math.pi)))
    table = []
    for jv in range(_WIN // _L):
        o = (iota + (jv * _L - 31)).astype(jnp.float32)
        z = o * jnp.float32(1.0 / _BW)
        table.append(coef * jnp.exp(jnp.float32(-0.5) * z * z))

    def fire_pair(jj):
        for r in range(2):
            b = b_base + jj * 2 + r
            off = ((jj % 2) * 2 + r) * _D
            pltpu.make_async_copy(
                x_hbm.at[b, t], row_v.at[pl.ds(off, _D)], sem).start()

    def drain_pair(jj):
        for r in range(2):
            b = b_base + jj * 2 + r
            off = ((jj % 2) * 2 + r) * _D
            pltpu.make_async_copy(
                x_hbm.at[b, t], row_v.at[pl.ds(off, _D)], sem).wait()

    fire_pair(0)

    minf_v = jnp.full((_L,), _NEG_INF, jnp.float32)
    big_v = jnp.full((_L,), _BIG_I32, jnp.int32)
    iota_u = [iota + u * _L for u in range(_U)]

    def fifth_lane_max(m):
        for _ in range(_N - 1):
            gm = jnp.max(m)
            m = jnp.where(m == gm, minf_v, m)
        return jnp.max(m)

    def select_row(cref, kcnt, o):
        ms = [minf_v] * _N
        is_ = [big_v] * _N
        nst = (kcnt + (_L - 1)) // _L

        def body(q, carry):
            ms = list(carry[:_N])
            is_ = list(carry[_N:])
            pos = q * _L + iota
            valid = pos < kcnt
            idx_raw = cref[pl.ds(q * _L, _L)]
            idx_safe = jnp.where(valid, idx_raw, 0)
            x = plsc.load_gather(row_v, [o + idx_safe])
            x = jnp.where(valid, x, minf_v)
            xi = jnp.where(valid, idx_raw, big_v)
            for k in range(_N):
                bsel = x > ms[k]
                nm = jnp.where(bsel, x, ms[k])
                ni = jnp.where(bsel, xi, is_[k])
                x = jnp.where(bsel, ms[k], x)
                xi = jnp.where(bsel, is_[k], xi)
                ms[k] = nm
                is_[k] = ni
            return tuple(ms) + tuple(is_)

        return lax.fori_loop(0, nst, body, tuple(ms) + tuple(is_))

    def extract_accumulate(state):
        ms = list(state[:_N])
        is_ = list(state[_N:])
        for _k in range(_N):
            gmax = jnp.max(ms[0])
            pm = ms[0] == gmax
            ibest = jnp.min(jnp.where(pm, is_[0], _BIG_I32))
            base = ibest + (_PAD - 31)
            for jv in range(_WIN // _L):
                idxv = base + (jv * _L) + iota
                plsc.addupdate_scatter(acc_v, [idxv], table[jv])
            pop = is_[0] == ibest
            for k in range(_N - 1):
                ms[k] = jnp.where(pop, ms[k + 1], ms[k])
                is_[k] = jnp.where(pop, is_[k + 1], is_[k])
            ms[_N - 1] = jnp.where(pop, jnp.full((_L,), _NEG_INF, jnp.float32),
                                   ms[_N - 1])
            is_[_N - 1] = jnp.where(pop, jnp.full((_L,), _BIG_I32, jnp.int32),
                                    is_[_N - 1])

    def pair_body(jj, _):
        drain_pair(jj)

        @pl.when(jj < 7)
        def _prefetch():
            fire_pair(jj + 1)

        slot_off = (jj % 2) * (2 * _D)
        oa = slot_off
        ob = slot_off + _D

        def p1(st, carry):
            ma, mb = carry
            base = st * (_U * _L)
            for u in range(_U):
                ma = jnp.maximum(ma, row_v[pl.ds(oa + base + u * _L, _L)])
                mb = jnp.maximum(mb, row_v[pl.ds(ob + base + u * _L, _L)])
            return ma, mb

        ma, mb = lax.fori_loop(0, _D // (_U * _L), p1, (minf_v, minf_v))
        ska, _sva = plsc.sort_key_val(ma, ma)
        skb, _svb = plsc.sort_key_val(mb, mb)
        t0a = jnp.squeeze(lax.slice(ska, (11,), (12,)))
        t0b = jnp.squeeze(lax.slice(skb, (11,), (12,)))

        def p2(st, carry):
            ca, cb = carry
            base = st * (_U * _L)
            news = []
            for o, t0, cref, cnt in ((oa, t0a, cand_a, ca),
                                     (ob, t0b, cand_b, cb)):
                xs = [row_v[pl.ds(o + base + u * _L, _L)] for u in range(_U)]
                msks = [x >= t0 for x in xs]
                for u in range(_U):
                    xi = iota_u[u] + base
                    plsc.store_compressed(
                        cand_buf_slice(cref, cnt), xi, mask=msks[u])
                    cnt = cnt + jnp.sum(msks[u], dtype=jnp.int32)
                news.append(cnt)
            return tuple(news)

        ka, kb = lax.fori_loop(0, _D // (_U * _L), p2,
                               (jnp.int32(0), jnp.int32(0)))

        sa = select_row(cand_a, ka, oa)
        sb = select_row(cand_b, kb, ob)
        extract_accumulate(sa)
        extract_accumulate(sb)
        return 0

    lax.fori_loop(0, 8, pair_body, 0)

    for cc in range(_D // 128):
        pltpu.make_async_copy(
            acc_v.at[pl.ds(_PAD + cc * 128, 128)],
            out_hbm.at[w, pl.ds(cc * 128, 128)], sem).start()
    for cc in range(_D // 128):
        pltpu.make_async_copy(
            acc_v.at[pl.ds(_PAD + cc * 128, 128)],
            out_hbm.at[w, pl.ds(cc * 128, 128)], sem).wait()


@jax.jit
def kernel(inputs):
    assert inputs.shape == (_B, _T, _D) and inputs.dtype == jnp.float32
    mesh = plsc.VectorSubcoreMesh(core_axis_name="c", subcore_axis_name="s")
    run = pl.kernel(
        _sc_kernel,
        out_type=jax.ShapeDtypeStruct((_B, _D), jnp.float32),
        mesh=mesh,
        scratch_types=[
            pltpu.VMEM((4 * _D,), jnp.float32),
            pltpu.VMEM((_ACC,), jnp.float32),
            pltpu.VMEM((_CAND,), jnp.int32),
            pltpu.VMEM((_CAND,), jnp.int32),
            pltpu.SemaphoreType.DMA,
        ],
        compiler_params=pltpu.CompilerParams(
            needs_layout_passes=False, use_tc_tiling_on_sc=True),
    )
    return run(inputs)


# speedup vs baseline: 1.4158x; 1.0501x over previous
import functools
import math

import jax
import jax.numpy as jnp
from jax import lax
from jax.experimental import pallas as pl
from jax.experimental.pallas import tpu as pltpu
from jax.experimental.pallas import tpu_sc as plsc

_B, _T, _D = 32, 16, 4096
_N = 5
_BW = 2.0
_L = 16
_WIN = 64
_PAD = 128
_ACC = _D + 2 * _PAD
_NEG_INF = float("-inf")
_BIG_I32 = 0x7FFFFFFF
_U = 16
_CAND = _D + _L


def cand_buf_slice(cref, cnt):
    return cref.at[pl.ds(cnt, _L)]


def _sc_kernel(x_hbm, out_hbm, row_v, acc_v, cand_a, cand_b, sem):
    c = lax.axis_index("c")
    s = lax.axis_index("s")
    w = s * 2 + c
    t = w // 2
    b_base = (w % 2) * 16

    iota = lax.iota(jnp.int32, _L)
    zeros = jnp.zeros((_L,), jnp.float32)


    coef = jnp.float32(1.0 / (_BW * math.sqrt(2.0 * math.pi)))
    table = []
    for jv in range(_WIN // _L):
        o = (iota + (jv * _L - 31)).astype(jnp.float32)
        z = o * jnp.float32(1.0 / _BW)
        table.append(coef * jnp.exp(jnp.float32(-0.5) * z * z))

    def fire_pair(jj):
        for r in range(2):
            b = b_base + jj * 2 + r
            off = ((jj % 2) * 2 + r) * _D
            pltpu.make_async_copy(
                x_hbm.at[b, t], row_v.at[pl.ds(off, _D)], sem).start()

    def drain_pair(jj):
        for r in range(2):
            b = b_base + jj * 2 + r
            off = ((jj % 2) * 2 + r) * _D
            pltpu.make_async_copy(
                x_hbm.at[b, t], row_v.at[pl.ds(off, _D)], sem).wait()

    fire_pair(0)

    def zero_body(i, _):
        acc_v[pl.ds(i * _L, _L)] = zeros
        return 0
    lax.fori_loop(0, _ACC // _L, zero_body, 0)

    minf_v = jnp.full((_L,), _NEG_INF, jnp.float32)
    big_v = jnp.full((_L,), _BIG_I32, jnp.int32)
    iota_u = [iota + u * _L for u in range(_U)]

    def fifth_lane_max(m):
        for _ in range(_N - 1):
            gm = jnp.max(m)
            m = jnp.where(m == gm, minf_v, m)
        return jnp.max(m)

    def select_row(cref, kcnt, o):
        ms = [minf_v] * _N
        is_ = [big_v] * _N
        nst = (kcnt + (_L - 1)) // _L

        def body(q, carry):
            ms = list(carry[:_N])
            is_ = list(carry[_N:])
            pos = q * _L + iota
            valid = pos < kcnt
            idx_raw = cref[pl.ds(q * _L, _L)]
            idx_safe = jnp.where(valid, idx_raw, 0)
            x = plsc.load_gather(row_v, [o + idx_safe])
            x = jnp.where(valid, x, minf_v)
            xi = jnp.where(valid, idx_raw, big_v)
            for k in range(_N):
                bsel = x > ms[k]
                nm = jnp.where(bsel, x, ms[k])
                ni = jnp.where(bsel, xi, is_[k])
                x = jnp.where(bsel, ms[k], x)
                xi = jnp.where(bsel, is_[k], xi)
                ms[k] = nm
                is_[k] = ni
            return tuple(ms) + tuple(is_)

        return lax.fori_loop(0, nst, body, tuple(ms) + tuple(is_))

    def extract_accumulate(state):
        ms = list(state[:_N])
        is_ = list(state[_N:])
        for _k in range(_N):
            gmax = jnp.max(ms[0])
            pm = ms[0] == gmax
            ibest = jnp.min(jnp.where(pm, is_[0], _BIG_I32))
            base = ibest + (_PAD - 31)
            for jv in range(_WIN // _L):
                idxv = base + (jv * _L) + iota
                plsc.addupdate_scatter(acc_v, [idxv], table[jv])
            pop = is_[0] == ibest
            for k in range(_N - 1):
                ms[k] = jnp.where(pop, ms[k + 1], ms[k])
                is_[k] = jnp.where(pop, is_[k + 1], is_[k])
            ms[_N - 1] = jnp.where(pop, jnp.full((_L,), _NEG_INF, jnp.float32),
                                   ms[_N - 1])
            is_[_N - 1] = jnp.where(pop, jnp.full((_L,), _BIG_I32, jnp.int32),
                                    is_[_N - 1])

    def pair_body(jj, _):
        drain_pair(jj)

        @pl.when(jj < 7)
        def _prefetch():
            fire_pair(jj + 1)

        slot_off = (jj % 2) * (2 * _D)
        oa = slot_off
        ob = slot_off + _D

        def p1(st, carry):
            ma, mb = carry
            base = st * (_U * _L)
            for u in range(_U):
                ma = jnp.maximum(ma, row_v[pl.ds(oa + base + u * _L, _L)])
                mb = jnp.maximum(mb, row_v[pl.ds(ob + base + u * _L, _L)])
            return ma, mb

        ma, mb = lax.fori_loop(0, _D // (_U * _L), p1, (minf_v, minf_v))
        ska, _sva = plsc.sort_key_val(ma, ma)
        skb, _svb = plsc.sort_key_val(mb, mb)
        t0a = jnp.squeeze(lax.slice(ska, (11,), (12,)))
        t0b = jnp.squeeze(lax.slice(skb, (11,), (12,)))

        def p2(st, carry):
            ca, cb = carry
            base = st * (_U * _L)
            news = []
            for o, t0, cref, cnt in ((oa, t0a, cand_a, ca),
                                     (ob, t0b, cand_b, cb)):
                xs = [row_v[pl.ds(o + base + u * _L, _L)] for u in range(_U)]
                msks = [x >= t0 for x in xs]
                for u in range(_U):
                    xi = iota_u[u] + base
                    plsc.store_compressed(
                        cand_buf_slice(cref, cnt), xi, mask=msks[u])
                    pc = plsc.all_reduce_population_count(msks[u])
                    cnt = cnt + jnp.squeeze(lax.slice(pc, (0,), (1,)))
                news.append(cnt)
            return tuple(news)

        ka, kb = lax.fori_loop(0, _D // (_U * _L), p2,
                               (jnp.int32(0), jnp.int32(0)))

        sa = select_row(cand_a, ka, oa)
        sb = select_row(cand_b, kb, ob)
        extract_accumulate(sa)
        extract_accumulate(sb)
        return 0

    lax.fori_loop(0, 8, pair_body, 0)

    for cc in range(_D // 128):
        pltpu.make_async_copy(
            acc_v.at[pl.ds(_PAD + cc * 128, 128)],
            out_hbm.at[w, pl.ds(cc * 128, 128)], sem).start()
    for cc in range(_D // 128):
        pltpu.make_async_copy(
            acc_v.at[pl.ds(_PAD + cc * 128, 128)],
            out_hbm.at[w, pl.ds(cc * 128, 128)], sem).wait()


@jax.jit
def kernel(inputs):
    assert inputs.shape == (_B, _T, _D) and inputs.dtype == jnp.float32
    mesh = plsc.VectorSubcoreMesh(core_axis_name="c", subcore_axis_name="s")
    run = pl.kernel(
        _sc_kernel,
        out_type=jax.ShapeDtypeStruct((_B, _D), jnp.float32),
        mesh=mesh,
        scratch_types=[
            pltpu.VMEM((4 * _D,), jnp.float32),
            pltpu.VMEM((_ACC,), jnp.float32),
            pltpu.VMEM((_CAND,), jnp.int32),
            pltpu.VMEM((_CAND,), jnp.int32),
            pltpu.SemaphoreType.DMA,
        ],
        compiler_params=pltpu.CompilerParams(
            needs_layout_passes=False, use_tc_tiling_on_sc=True),
    )
    return run(inputs)
